# baseline (speedup 1.0000x reference)
import functools
import math

import jax
import jax.numpy as jnp
import numpy as np
from jax import lax
from jax.experimental import pallas as pl
from jax.experimental.pallas import tpu as pltpu
from jax.experimental.pallas import tpu_sc as plsc

ATOM_DIMS = [119, 4, 12, 12, 10, 6, 6, 2, 2]
BOND_DIMS = [5, 6, 2]
LATENT = 128
N, E, F, G = 10000, 160000, 20000, 100
NPG, EPG, FPG = N // G, E // G, F // G
DROPNET = 0.1


def _dot(a, b):
    return lax.dot_general(a, b, (((1,), (0,)), ((), ())),
                           precision=lax.Precision.HIGHEST,
                           preferred_element_type=jnp.float32)


def _dot_hi(a, b):
    return lax.dot_general(a, b, (((1,), (0,)), ((), ())),
                           precision=lax.Precision.HIGHEST,
                           preferred_element_type=jnp.float32)


def _enc_body(xf_ref, wd_ref, c_ref, w2_ref, b2_ref, o_ref):
    h = jax.nn.relu(_dot_hi(xf_ref[...], wd_ref[...]) + c_ref[...])
    o_ref[...] = _dot(h, w2_ref[...]) + b2_ref[...]


def _encode(xf, wd, c, w2, b2, rows_per_blk):
    n = xf.shape[0]
    grid = n // rows_per_blk
    return pl.pallas_call(
        _enc_body,
        grid=(grid,),
        in_specs=[
            pl.BlockSpec((rows_per_blk, xf.shape[1]), lambda i: (i, 0)),
            pl.BlockSpec(wd.shape, lambda i: (0, 0)),
            pl.BlockSpec((1, LATENT), lambda i: (0, 0)),
            pl.BlockSpec((LATENT, LATENT), lambda i: (0, 0)),
            pl.BlockSpec((1, LATENT), lambda i: (0, 0)),
        ],
        out_specs=pl.BlockSpec((rows_per_blk, LATENT), lambda i: (i, 0)),
        out_shape=jax.ShapeDtypeStruct((n, LATENT), jnp.float32),
    )(xf, wd, c, w2, b2)


def _proj2_body(x_ref, w_ref, o1_ref, o2_ref):
    t = _dot(x_ref[...], w_ref[...])
    o1_ref[...] = t[:, :LATENT]
    o2_ref[...] = t[:, LATENT:]


def _proj2(x, w2x, rows_per_blk):
    n = x.shape[0]
    grid = n // rows_per_blk
    out = jax.ShapeDtypeStruct((n, LATENT), jnp.float32)
    return pl.pallas_call(
        _proj2_body,
        grid=(grid,),
        in_specs=[
            pl.BlockSpec((rows_per_blk, LATENT), lambda i: (i, 0)),
            pl.BlockSpec((LATENT, 2 * LATENT), lambda i: (0, 0)),
        ],
        out_specs=[pl.BlockSpec((rows_per_blk, LATENT), lambda i: (i, 0))] * 2,
        out_shape=[out, out],
    )(x, w2x)


def _mm_bias_body(x_ref, w_ref, b_ref, o_ref):
    o_ref[...] = _dot(x_ref[...], w_ref[...]) + b_ref[...]


def _mm_bias(x, w, b):
    return pl.pallas_call(
        _mm_bias_body,
        in_specs=[pl.BlockSpec(x.shape, lambda: (0, 0)),
                  pl.BlockSpec(w.shape, lambda: (0, 0)),
                  pl.BlockSpec((1, w.shape[1]), lambda: (0, 0))],
        out_specs=pl.BlockSpec((x.shape[0], w.shape[1]), lambda: (0, 0)),
        out_shape=jax.ShapeDtypeStruct((x.shape[0], w.shape[1]), jnp.float32),
    )(x, w, b.reshape(1, -1))


def _edge_fin_body(hpre_ref, ea_ref, a_ref, w2_ref, ue_ref, b2_ref,
                   e1_ref, eanew_ref, eag_ref):
    t = _dot(ea_ref[0], a_ref[...])
    h = jax.nn.relu(hpre_ref[0] + t + ue_ref[0])
    e1 = _dot(h, w2_ref[...]) + b2_ref[...]
    e1_ref[0] = e1
    eanew_ref[0] = ea_ref[0] + e1
    eag_ref[0] = jnp.sum(e1, axis=0, keepdims=True)


def _edge_finish(hpre, ea, a, w2, ue_tbl, b2):
    outs = [jax.ShapeDtypeStruct((G, EPG, LATENT), jnp.float32),
            jax.ShapeDtypeStruct((G, EPG, LATENT), jnp.float32),
            jax.ShapeDtypeStruct((G, 1, LATENT), jnp.float32)]
    eblk = pl.BlockSpec((1, EPG, LATENT), lambda g: (g, 0, 0))
    e1, eanew, eag = pl.pallas_call(
        _edge_fin_body,
        grid=(G,),
        in_specs=[
            eblk, eblk,
            pl.BlockSpec((LATENT, LATENT), lambda g: (0, 0)),
            pl.BlockSpec((LATENT, LATENT), lambda g: (0, 0)),
            pl.BlockSpec((1, 1, LATENT), lambda g: (g, 0, 0)),
            pl.BlockSpec((1, LATENT), lambda g: (0, 0)),
        ],
        out_specs=[eblk, eblk,
                   pl.BlockSpec((1, 1, LATENT), lambda g: (g, 0, 0))],
        out_shape=outs,
    )(hpre.reshape(G, EPG, LATENT), ea.reshape(G, EPG, LATENT), a, w2,
      ue_tbl.reshape(G, 1, LATENT), b2)
    return (e1.reshape(E, LATENT), eanew.reshape(E, LATENT),
            eag.reshape(G, LATENT))


def _upd_body(x_ref, s_ref, r_ref, wa_ref, wb_ref, wc_ref, g_ref, w2_ref,
              b2_ref, xnew_ref, agg_ref, xsum_ref):
    h = jax.nn.relu(_dot(x_ref[0], wa_ref[...]) + _dot(s_ref[0], wb_ref[...])
                    + _dot(r_ref[0], wc_ref[...]) + g_ref[0])
    x1 = _dot(h, w2_ref[...]) + b2_ref[...]
    xnew = x_ref[0] + x1
    xnew_ref[0] = xnew
    agg_ref[0] = jnp.sum(x1, axis=0, keepdims=True)
    xsum_ref[0] = jnp.sum(xnew, axis=0, keepdims=True)


def _update(x, scat, s_off, r_off, wa, wb, wc, g_tbl, w2, b2, rows):
    n = x.shape[0]
    grid = n // rows
    scat3 = scat.reshape(-1, rows, LATENT)
    outs = [jax.ShapeDtypeStruct((grid, rows, LATENT), jnp.float32),
            jax.ShapeDtypeStruct((G, 1, LATENT), jnp.float32),
            jax.ShapeDtypeStruct((G, 1, LATENT), jnp.float32)]
    blk = pl.BlockSpec((1, rows, LATENT), lambda g: (g, 0, 0))
    mk = lambda off: pl.BlockSpec((1, rows, LATENT),
                                  lambda g, off=off: (g + off, 0, 0))
    wspec = pl.BlockSpec((LATENT, LATENT), lambda g: (0, 0))
    row3 = pl.BlockSpec((1, 1, LATENT), lambda g: (g, 0, 0))
    xnew, agg, xsum = pl.pallas_call(
        _upd_body,
        grid=(grid,),
        in_specs=[
            blk,
            mk(s_off), mk(r_off),
            wspec, wspec, wspec,
            row3,
            wspec,
            pl.BlockSpec((1, LATENT), lambda g: (0, 0)),
        ],
        out_specs=[blk, row3, row3],
        out_shape=outs,
    )(x.reshape(grid, rows, LATENT), scat3, scat3, wa, wb, wc,
      g_tbl.reshape(G, 1, LATENT), w2, b2)
    return (xnew.reshape(n, LATENT), agg.reshape(G, LATENT),
            xsum.reshape(G, LATENT))


def _glob_body(u_ref, na_ref, ea_ref, fa_ref, wa_ref, wb_ref, wc_ref, wd_ref,
               b1_ref, w2_ref, b2_ref, unew_ref):
    h = jax.nn.relu(_dot(u_ref[...], wa_ref[...]) + _dot(na_ref[...], wb_ref[...])
                    + _dot(ea_ref[...], wc_ref[...]) + _dot(fa_ref[...], wd_ref[...])
                    + b1_ref[...])
    unew_ref[...] = u_ref[...] + _dot(h, w2_ref[...]) + b2_ref[...]


def _glob_update(u, na, eag, fa, wa, wb, wc, wd, b1, w2, b2):
    full = lambda s: pl.BlockSpec(s, lambda: tuple(0 for _ in s))
    return pl.pallas_call(
        _glob_body,
        in_specs=[full((G, LATENT))] * 4 + [full((LATENT, LATENT))] * 4
        + [full((1, LATENT)), full((LATENT, LATENT)), full((1, LATENT))],
        out_specs=full((G, LATENT)),
        out_shape=jax.ShapeDtypeStruct((G, LATENT), jnp.float32),
    )(u, na, eag, fa, wa, wb, wc, wd, b1.reshape(1, -1), w2, b2.reshape(1, -1))


def _dec_body(pxm_ref, pfm_ref, w1a_ref, w1b_ref, b1_ref, w2_ref, b2_ref, o_ref):
    h = jax.nn.relu(_dot(pxm_ref[...], w1a_ref[...]) + _dot(pfm_ref[...], w1b_ref[...])
                    + b1_ref[...])
    o_ref[...] = _dot(h, w2_ref[...]) + b2_ref[...]


def _decode(pxm, pfm, w1a, w1b, b1, w2p, b2p):
    full = lambda s: pl.BlockSpec(s, lambda: tuple(0 for _ in s))
    return pl.pallas_call(
        _dec_body,
        in_specs=[full((G, LATENT))] * 2 + [full((LATENT, LATENT))] * 2
        + [full((1, LATENT)), full((LATENT, LATENT)), full((1, LATENT))],
        out_specs=full((G, LATENT)),
        out_shape=jax.ShapeDtypeStruct((G, LATENT), jnp.float32),
    )(pxm, pfm, w1a, w1b, b1.reshape(1, -1), w2p, b2p)


def _const_body(bg1_ref, wg2_ref, bg2_ref, bf1_ref, wf2_ref, bf2_ref,
                u0_ref, f0_ref):
    u0_ref[...] = _dot(jax.nn.relu(bg1_ref[...]), wg2_ref[...]) + bg2_ref[...]
    f0_ref[...] = _dot(jax.nn.relu(bf1_ref[...]), wf2_ref[...]) + bf2_ref[...]


def _consts(bg1, wg2, bg2, bf1, wf2, bf2):
    full = lambda s: pl.BlockSpec(s, lambda: tuple(0 for _ in s))
    out = jax.ShapeDtypeStruct((1, LATENT), jnp.float32)
    return pl.pallas_call(
        _const_body,
        in_specs=[full((1, LATENT)), full((LATENT, LATENT)), full((1, LATENT))] * 2,
        out_specs=[full((1, LATENT))] * 2,
        out_shape=[out, out],
    )(bg1.reshape(1, -1), wg2, bg2.reshape(1, -1),
      bf1.reshape(1, -1), wf2, bf2.reshape(1, -1))


_NC, _NS, _NW = 2, 16, 32
_C = 128
_NCH = E // _C
_NG = LATENT // 16


@functools.lru_cache(maxsize=None)
def _make_gather(K):
    NB = K
    mesh = plsc.VectorSubcoreMesh(core_axis_name="c", subcore_axis_name="s")

    @functools.partial(
        pl.kernel, mesh=mesh,
        out_type=jax.ShapeDtypeStruct((E, LATENT), jnp.float32),
        scratch_types=(
            [pltpu.VMEM((K, _C), jnp.int32)]
            + [pltpu.VMEM((_C, LATENT), jnp.float32) for _ in range(NB)]
            + [pltpu.SemaphoreType.DMA for _ in range(NB)]),
    )
    def k(tbl_hbm, idx_hbm, out_hbm, idxbuf, *rest):
        gbufs = rest[:NB]
        sems = rest[NB:]
        c = lax.axis_index("c")
        s = lax.axis_index("s")
        wid = s * _NC + c
        per = _NCH // _NW
        nj = per + (wid < _NCH - per * _NW).astype(jnp.int32)

        def add_into(dst, src):
            def body(r4, _):
                for u in range(4):
                    r = r4 * 4 + u
                    for g in range(_NG):
                        sl = pl.ds(g * 16, 16)
                        plsc.addupdate(dst.at[r, sl], src[r, sl])
                return 0
            lax.fori_loop(0, _C // 4, body, 0)

        def chunk(j, _):
            ci = j * _NW + wid
            off = ci * _C
            pltpu.sync_copy(idx_hbm.at[:, ci, :], idxbuf)
            hs = [pltpu.async_copy(tbl_hbm.at[idxbuf.at[t]], gbufs[t], sems[t])
                  for t in range(NB)]
            hs[0].wait()
            for t in range(1, K):
                hs[t].wait()
                add_into(gbufs[0], gbufs[t])
            pltpu.sync_copy(gbufs[0], out_hbm.at[pl.ds(off, _C)])
            return 0
        lax.fori_loop(0, nj, chunk, 0)

    return k


def _gather_sum(tbl, idxs):
    K = idxs.shape[0]
    return _make_gather(K)(tbl, idxs.reshape(K, _NCH, _C))


def _scatter_sum(e1, idxs, nrows):
    acc = jnp.zeros((nrows, LATENT), jnp.float32)
    dd = jnp.concatenate([e1, e1], axis=0)
    acc = acc.at[jnp.concatenate([idxs[0], idxs[1]])].add(dd)
    acc = acc.at[jnp.concatenate([idxs[2], idxs[3]])].add(dd)
    return acc


def kernel(x, edge_index, edge_attr, node_batch, face_mask, face_index,
           num_nodes, num_faces, num_edges, params):
    P = params
    row, col = edge_index[0], edge_index[1]
    fi0, fi1 = face_index[0], face_index[1]
    idxs4 = jnp.stack([row, col + N, fi0 + 2 * N, fi1 + 2 * N + F])
    idxs2 = idxs4[:2]

    offs_a = np.concatenate([[0], np.cumsum(ATOM_DIMS)[:-1]])
    offs_b = np.concatenate([[0], np.cumsum(BOND_DIMS)[:-1]])

    Wn1 = P["enc_node"]["W"][0].astype(jnp.bfloat16).astype(jnp.float32)
    cn = (P["enc_node"]["b"][0] + Wn1[offs_a].sum(0)).reshape(1, -1)
    Wdn = Wn1[offs_a + 1] - Wn1[offs_a]
    xf = jnp.pad(x.astype(jnp.float32), ((0, 0), (0, 7)))
    Wdnp = jnp.pad(Wdn, ((0, 7), (0, 0)))
    xl = _encode(xf, Wdnp, cn, P["enc_node"]["W"][1],
                 P["enc_node"]["b"][1].reshape(1, -1), 2000)

    We1 = P["enc_edge"]["W"][0].astype(jnp.bfloat16).astype(jnp.float32)
    ce = (P["enc_edge"]["b"][0] + We1[offs_b].sum(0)).reshape(1, -1)
    Wde = We1[offs_b + 1] - We1[offs_b]
    eaf = jnp.pad(edge_attr.astype(jnp.float32), ((0, 0), (0, 5)))
    Wdep = jnp.pad(Wde, ((0, 5), (0, 0)))
    ea = _encode(eaf, Wdep, ce, P["enc_edge"]["W"][1],
                 P["enc_edge"]["b"][1].reshape(1, -1), 4000)

    u0, f0 = _consts(P["enc_global"]["b"][0], P["enc_global"]["W"][1],
                     P["enc_global"]["b"][1], P["enc_face"]["b"][0],
                     P["enc_face"]["W"][1], P["enc_face"]["b"][1])
    u = jnp.broadcast_to(u0, (G, LATENT))
    face = jnp.broadcast_to(f0, (F, LATENT))

    for li, lp in enumerate(P["layers"]):
        W = lp["edge"]["W"][0]
        A = W[:LATENT]
        Wbc = jnp.concatenate([W[LATENT:2 * LATENT], W[2 * LATENT:3 * LATENT]], axis=1)
        D = W[3 * LATENT:4 * LATENT]
        Wef = jnp.concatenate([W[4 * LATENT:5 * LATENT], W[5 * LATENT:]], axis=1)

        xb, xc = _proj2(xl, Wbc, 2000)
        Wu = jnp.concatenate([D, lp["node"]["W"][0][3 * LATENT:],
                              lp["face"]["W"][0][3 * LATENT:]], axis=1)
        bu = jnp.concatenate([lp["edge"]["b"][0], lp["node"]["b"][0],
                              lp["face"]["b"][0]])
        uproj = _mm_bias(u, Wu, bu)
        ue_tbl = uproj[:, :LATENT]
        gn_tbl = uproj[:, LATENT:2 * LATENT]
        gf_tbl = uproj[:, 2 * LATENT:]

        if li == 0:
            fef = _dot(f0, Wef)
            ue_tbl = ue_tbl + fef[:, :LATENT] + fef[:, LATENT:]
            tbl = jnp.concatenate([xb, xc], axis=0)
            hpre = _gather_sum(tbl, idxs2)
        else:
            fe, ff = _proj2(face, Wef, 2000)
            tbl = jnp.concatenate([xb, xc, fe, ff], axis=0)
            hpre = _gather_sum(tbl, idxs4)

        e1, ea, eag = _edge_finish(hpre, ea, A, lp["edge"]["W"][1],
                                   ue_tbl, lp["edge"]["b"][1].reshape(1, -1))

        scat = _scatter_sum(e1, idxs4, 2 * (N + F))

        Wn = lp["node"]["W"][0]
        xl, na, xsum = _update(xl, scat, 0, N // NPG,
                               Wn[:LATENT], Wn[LATENT:2 * LATENT],
                               Wn[2 * LATENT:3 * LATENT], gn_tbl,
                               lp["node"]["W"][1],
                               lp["node"]["b"][1].reshape(1, -1), NPG)
        Wf = lp["face"]["W"][0]
        face, fa, fsum = _update(face, scat, 2 * N // FPG, (2 * N + F) // FPG,
                                 Wf[:LATENT], Wf[LATENT:2 * LATENT],
                                 Wf[2 * LATENT:3 * LATENT], gf_tbl,
                                 lp["face"]["W"][1],
                                 lp["face"]["b"][1].reshape(1, -1), FPG)
        Wg = lp["glob"]["W"][0]
        u = _glob_update(u, na, eag, fa, Wg[:LATENT], Wg[LATENT:2 * LATENT],
                         Wg[2 * LATENT:3 * LATENT], Wg[3 * LATENT:],
                         lp["glob"]["b"][0], lp["glob"]["W"][1],
                         lp["glob"]["b"][1])

    r1 = jax.random.uniform(jax.random.key(7), (G, 1), dtype=jnp.float32)
    m1 = ((r1 >= DROPNET) | (num_faces[:, None] == 1)).astype(jnp.float32)
    m2 = ((1.0 - r1) >= DROPNET).astype(jnp.float32)
    pxm = xsum * m1
    pfm = fsum * m2
    Wd = P["decoder"]["W"][0]
    w2p = jnp.pad(P["decoder"]["W"][1], ((0, 0), (0, LATENT - 1)))
    b2p = jnp.pad(P["decoder"]["b"][1], (0, LATENT - 1)).reshape(1, -1)
    outp = _decode(pxm, pfm, Wd[:LATENT], Wd[LATENT:],
                   P["decoder"]["b"][0], w2p, b2p)
    return outp[:, :1]


# speedup vs baseline: 1.0502x; 1.0502x over previous
import functools
import math

import jax
import jax.numpy as jnp
import numpy as np
from jax import lax
from jax.experimental import pallas as pl
from jax.experimental.pallas import tpu as pltpu
from jax.experimental.pallas import tpu_sc as plsc

ATOM_DIMS = [119, 4, 12, 12, 10, 6, 6, 2, 2]
BOND_DIMS = [5, 6, 2]
LATENT = 128
N, E, F, G = 10000, 160000, 20000, 100
NPG, EPG, FPG = N // G, E // G, F // G
DROPNET = 0.1


def _dot(a, b):
    return lax.dot_general(a, b, (((1,), (0,)), ((), ())),
                           precision=lax.Precision.HIGHEST,
                           preferred_element_type=jnp.float32)


def _dot_hi(a, b):
    return lax.dot_general(a, b, (((1,), (0,)), ((), ())),
                           precision=lax.Precision.HIGHEST,
                           preferred_element_type=jnp.float32)


def _enc_body(xf_ref, wd_ref, c_ref, w2_ref, b2_ref, o_ref):
    h = jax.nn.relu(_dot_hi(xf_ref[...], wd_ref[...]) + c_ref[...])
    o_ref[...] = _dot(h, w2_ref[...]) + b2_ref[...]


def _encode(xf, wd, c, w2, b2, rows_per_blk):
    n = xf.shape[0]
    grid = n // rows_per_blk
    return pl.pallas_call(
        _enc_body,
        grid=(grid,),
        in_specs=[
            pl.BlockSpec((rows_per_blk, xf.shape[1]), lambda i: (i, 0)),
            pl.BlockSpec(wd.shape, lambda i: (0, 0)),
            pl.BlockSpec((1, LATENT), lambda i: (0, 0)),
            pl.BlockSpec((LATENT, LATENT), lambda i: (0, 0)),
            pl.BlockSpec((1, LATENT), lambda i: (0, 0)),
        ],
        out_specs=pl.BlockSpec((rows_per_blk, LATENT), lambda i: (i, 0)),
        out_shape=jax.ShapeDtypeStruct((n, LATENT), jnp.float32),
    )(xf, wd, c, w2, b2)


def _proj2_body(x_ref, w_ref, o1_ref, o2_ref):
    t = _dot(x_ref[...], w_ref[...])
    o1_ref[...] = t[:, :LATENT]
    o2_ref[...] = t[:, LATENT:]


def _proj2(x, w2x, rows_per_blk):
    n = x.shape[0]
    grid = n // rows_per_blk
    out = jax.ShapeDtypeStruct((n, LATENT), jnp.float32)
    return pl.pallas_call(
        _proj2_body,
        grid=(grid,),
        in_specs=[
            pl.BlockSpec((rows_per_blk, LATENT), lambda i: (i, 0)),
            pl.BlockSpec((LATENT, 2 * LATENT), lambda i: (0, 0)),
        ],
        out_specs=[pl.BlockSpec((rows_per_blk, LATENT), lambda i: (i, 0))] * 2,
        out_shape=[out, out],
    )(x, w2x)


def _mm_bias_body(x_ref, w_ref, b_ref, o_ref):
    o_ref[...] = _dot(x_ref[...], w_ref[...]) + b_ref[...]


def _mm_bias(x, w, b):
    return pl.pallas_call(
        _mm_bias_body,
        in_specs=[pl.BlockSpec(x.shape, lambda: (0, 0)),
                  pl.BlockSpec(w.shape, lambda: (0, 0)),
                  pl.BlockSpec((1, w.shape[1]), lambda: (0, 0))],
        out_specs=pl.BlockSpec((x.shape[0], w.shape[1]), lambda: (0, 0)),
        out_shape=jax.ShapeDtypeStruct((x.shape[0], w.shape[1]), jnp.float32),
    )(x, w, b.reshape(1, -1))


def _edge_fin_body(hpre_ref, ea_ref, a_ref, w2_ref, ue_ref, b2_ref,
                   e1_ref, eanew_ref, eag_ref):
    t = _dot(ea_ref[0], a_ref[...])
    h = jax.nn.relu(hpre_ref[0] + t + ue_ref[0])
    e1 = _dot(h, w2_ref[...]) + b2_ref[...]
    e1_ref[0] = e1
    eanew_ref[0] = ea_ref[0] + e1
    eag_ref[0] = jnp.sum(e1, axis=0, keepdims=True)


def _edge_finish(hpre, ea, a, w2, ue_tbl, b2):
    outs = [jax.ShapeDtypeStruct((G, EPG, LATENT), jnp.float32),
            jax.ShapeDtypeStruct((G, EPG, LATENT), jnp.float32),
            jax.ShapeDtypeStruct((G, 1, LATENT), jnp.float32)]
    eblk = pl.BlockSpec((1, EPG, LATENT), lambda g: (g, 0, 0))
    e1, eanew, eag = pl.pallas_call(
        _edge_fin_body,
        grid=(G,),
        in_specs=[
            eblk, eblk,
            pl.BlockSpec((LATENT, LATENT), lambda g: (0, 0)),
            pl.BlockSpec((LATENT, LATENT), lambda g: (0, 0)),
            pl.BlockSpec((1, 1, LATENT), lambda g: (g, 0, 0)),
            pl.BlockSpec((1, LATENT), lambda g: (0, 0)),
        ],
        out_specs=[eblk, eblk,
                   pl.BlockSpec((1, 1, LATENT), lambda g: (g, 0, 0))],
        out_shape=outs,
    )(hpre.reshape(G, EPG, LATENT), ea.reshape(G, EPG, LATENT), a, w2,
      ue_tbl.reshape(G, 1, LATENT), b2)
    return (e1.reshape(E, LATENT), eanew.reshape(E, LATENT),
            eag.reshape(G, LATENT))


def _upd_body(x_ref, s_ref, r_ref, wa_ref, wb_ref, wc_ref, g_ref, w2_ref,
              b2_ref, xnew_ref, agg_ref, xsum_ref):
    h = jax.nn.relu(_dot(x_ref[0], wa_ref[...]) + _dot(s_ref[0], wb_ref[...])
                    + _dot(r_ref[0], wc_ref[...]) + g_ref[0])
    x1 = _dot(h, w2_ref[...]) + b2_ref[...]
    xnew = x_ref[0] + x1
    xnew_ref[0] = xnew
    agg_ref[0] = jnp.sum(x1, axis=0, keepdims=True)
    xsum_ref[0] = jnp.sum(xnew, axis=0, keepdims=True)


def _update(x, scat, s_off, r_off, wa, wb, wc, g_tbl, w2, b2, rows):
    n = x.shape[0]
    grid = n // rows
    scat3 = scat.reshape(-1, rows, LATENT)
    outs = [jax.ShapeDtypeStruct((grid, rows, LATENT), jnp.float32),
            jax.ShapeDtypeStruct((G, 1, LATENT), jnp.float32),
            jax.ShapeDtypeStruct((G, 1, LATENT), jnp.float32)]
    blk = pl.BlockSpec((1, rows, LATENT), lambda g: (g, 0, 0))
    mk = lambda off: pl.BlockSpec((1, rows, LATENT),
                                  lambda g, off=off: (g + off, 0, 0))
    wspec = pl.BlockSpec((LATENT, LATENT), lambda g: (0, 0))
    row3 = pl.BlockSpec((1, 1, LATENT), lambda g: (g, 0, 0))
    xnew, agg, xsum = pl.pallas_call(
        _upd_body,
        grid=(grid,),
        in_specs=[
            blk,
            mk(s_off), mk(r_off),
            wspec, wspec, wspec,
            row3,
            wspec,
            pl.BlockSpec((1, LATENT), lambda g: (0, 0)),
        ],
        out_specs=[blk, row3, row3],
        out_shape=outs,
    )(x.reshape(grid, rows, LATENT), scat3, scat3, wa, wb, wc,
      g_tbl.reshape(G, 1, LATENT), w2, b2)
    return (xnew.reshape(n, LATENT), agg.reshape(G, LATENT),
            xsum.reshape(G, LATENT))


def _glob_body(u_ref, na_ref, ea_ref, fa_ref, wa_ref, wb_ref, wc_ref, wd_ref,
               b1_ref, w2_ref, b2_ref, unew_ref):
    h = jax.nn.relu(_dot(u_ref[...], wa_ref[...]) + _dot(na_ref[...], wb_ref[...])
                    + _dot(ea_ref[...], wc_ref[...]) + _dot(fa_ref[...], wd_ref[...])
                    + b1_ref[...])
    unew_ref[...] = u_ref[...] + _dot(h, w2_ref[...]) + b2_ref[...]


def _glob_update(u, na, eag, fa, wa, wb, wc, wd, b1, w2, b2):
    full = lambda s: pl.BlockSpec(s, lambda: tuple(0 for _ in s))
    return pl.pallas_call(
        _glob_body,
        in_specs=[full((G, LATENT))] * 4 + [full((LATENT, LATENT))] * 4
        + [full((1, LATENT)), full((LATENT, LATENT)), full((1, LATENT))],
        out_specs=full((G, LATENT)),
        out_shape=jax.ShapeDtypeStruct((G, LATENT), jnp.float32),
    )(u, na, eag, fa, wa, wb, wc, wd, b1.reshape(1, -1), w2, b2.reshape(1, -1))


def _dec_body(pxm_ref, pfm_ref, w1a_ref, w1b_ref, b1_ref, w2_ref, b2_ref, o_ref):
    h = jax.nn.relu(_dot(pxm_ref[...], w1a_ref[...]) + _dot(pfm_ref[...], w1b_ref[...])
                    + b1_ref[...])
    o_ref[...] = _dot(h, w2_ref[...]) + b2_ref[...]


def _decode(pxm, pfm, w1a, w1b, b1, w2p, b2p):
    full = lambda s: pl.BlockSpec(s, lambda: tuple(0 for _ in s))
    return pl.pallas_call(
        _dec_body,
        in_specs=[full((G, LATENT))] * 2 + [full((LATENT, LATENT))] * 2
        + [full((1, LATENT)), full((LATENT, LATENT)), full((1, LATENT))],
        out_specs=full((G, LATENT)),
        out_shape=jax.ShapeDtypeStruct((G, LATENT), jnp.float32),
    )(pxm, pfm, w1a, w1b, b1.reshape(1, -1), w2p, b2p)


def _const_body(bg1_ref, wg2_ref, bg2_ref, bf1_ref, wf2_ref, bf2_ref,
                u0_ref, f0_ref):
    u0_ref[...] = _dot(jax.nn.relu(bg1_ref[...]), wg2_ref[...]) + bg2_ref[...]
    f0_ref[...] = _dot(jax.nn.relu(bf1_ref[...]), wf2_ref[...]) + bf2_ref[...]


def _consts(bg1, wg2, bg2, bf1, wf2, bf2):
    full = lambda s: pl.BlockSpec(s, lambda: tuple(0 for _ in s))
    out = jax.ShapeDtypeStruct((1, LATENT), jnp.float32)
    return pl.pallas_call(
        _const_body,
        in_specs=[full((1, LATENT)), full((LATENT, LATENT)), full((1, LATENT))] * 2,
        out_specs=[full((1, LATENT))] * 2,
        out_shape=[out, out],
    )(bg1.reshape(1, -1), wg2, bg2.reshape(1, -1),
      bf1.reshape(1, -1), wf2, bf2.reshape(1, -1))


_NC, _NS, _NW = 2, 16, 32
_C = 128
_NCH = E // _C
_NG = LATENT // 16


@functools.lru_cache(maxsize=None)
def _make_gather(K):
    NB = K
    mesh = plsc.VectorSubcoreMesh(core_axis_name="c", subcore_axis_name="s")

    @functools.partial(
        pl.kernel, mesh=mesh,
        out_type=jax.ShapeDtypeStruct((E, LATENT), jnp.float32),
        scratch_types=(
            [pltpu.VMEM((K, _C), jnp.int32)]
            + [pltpu.VMEM((_C, LATENT), jnp.float32) for _ in range(NB)]
            + [pltpu.SemaphoreType.DMA for _ in range(NB)]),
    )
    def k(tbl_hbm, idx_hbm, out_hbm, idxbuf, *rest):
        gbufs = rest[:NB]
        sems = rest[NB:]
        c = lax.axis_index("c")
        s = lax.axis_index("s")
        wid = s * _NC + c
        per = _NCH // _NW
        nj = per + (wid < _NCH - per * _NW).astype(jnp.int32)

        def add_into(dst, src):
            def body(r4, _):
                for u in range(4):
                    r = r4 * 4 + u
                    for g in range(_NG):
                        sl = pl.ds(g * 16, 16)
                        plsc.addupdate(dst.at[r, sl], src[r, sl])
                return 0
            lax.fori_loop(0, _C // 4, body, 0)

        def chunk(j, _):
            ci = j * _NW + wid
            off = ci * _C
            pltpu.sync_copy(idx_hbm.at[:, ci, :], idxbuf)
            hs = [pltpu.async_copy(tbl_hbm.at[idxbuf.at[t]], gbufs[t], sems[t])
                  for t in range(NB)]
            hs[0].wait()
            for t in range(1, K):
                hs[t].wait()
                add_into(gbufs[0], gbufs[t])
            pltpu.sync_copy(gbufs[0], out_hbm.at[pl.ds(off, _C)])
            return 0
        lax.fori_loop(0, nj, chunk, 0)

    return k


def _gather_sum(tbl, idxs):
    K = idxs.shape[0]
    return _make_gather(K)(tbl, idxs.reshape(K, _NCH, _C))


def _scatter_sum(e1, idxs, nrows):
    acc = jnp.zeros((nrows, LATENT), jnp.float32)
    for t in range(idxs.shape[0]):
        acc = acc.at[idxs[t]].add(e1)
    return acc


def kernel(x, edge_index, edge_attr, node_batch, face_mask, face_index,
           num_nodes, num_faces, num_edges, params):
    P = params
    row, col = edge_index[0], edge_index[1]
    fi0, fi1 = face_index[0], face_index[1]
    idxs4 = jnp.stack([row, col + N, fi0 + 2 * N, fi1 + 2 * N + F])
    idxs2 = idxs4[:2]

    offs_a = np.concatenate([[0], np.cumsum(ATOM_DIMS)[:-1]])
    offs_b = np.concatenate([[0], np.cumsum(BOND_DIMS)[:-1]])

    Wn1 = P["enc_node"]["W"][0].astype(jnp.bfloat16).astype(jnp.float32)
    cn = (P["enc_node"]["b"][0] + Wn1[offs_a].sum(0)).reshape(1, -1)
    Wdn = Wn1[offs_a + 1] - Wn1[offs_a]
    xf = jnp.pad(x.astype(jnp.float32), ((0, 0), (0, 7)))
    Wdnp = jnp.pad(Wdn, ((0, 7), (0, 0)))
    xl = _encode(xf, Wdnp, cn, P["enc_node"]["W"][1],
                 P["enc_node"]["b"][1].reshape(1, -1), 2000)

    We1 = P["enc_edge"]["W"][0].astype(jnp.bfloat16).astype(jnp.float32)
    ce = (P["enc_edge"]["b"][0] + We1[offs_b].sum(0)).reshape(1, -1)
    Wde = We1[offs_b + 1] - We1[offs_b]
    eaf = jnp.pad(edge_attr.astype(jnp.float32), ((0, 0), (0, 5)))
    Wdep = jnp.pad(Wde, ((0, 5), (0, 0)))
    ea = _encode(eaf, Wdep, ce, P["enc_edge"]["W"][1],
                 P["enc_edge"]["b"][1].reshape(1, -1), 4000)

    u0, f0 = _consts(P["enc_global"]["b"][0], P["enc_global"]["W"][1],
                     P["enc_global"]["b"][1], P["enc_face"]["b"][0],
                     P["enc_face"]["W"][1], P["enc_face"]["b"][1])
    u = jnp.broadcast_to(u0, (G, LATENT))
    face = jnp.broadcast_to(f0, (F, LATENT))

    for li, lp in enumerate(P["layers"]):
        W = lp["edge"]["W"][0]
        A = W[:LATENT]
        Wbc = jnp.concatenate([W[LATENT:2 * LATENT], W[2 * LATENT:3 * LATENT]], axis=1)
        D = W[3 * LATENT:4 * LATENT]
        Wef = jnp.concatenate([W[4 * LATENT:5 * LATENT], W[5 * LATENT:]], axis=1)

        xb, xc = _proj2(xl, Wbc, 2000)
        Wu = jnp.concatenate([D, lp["node"]["W"][0][3 * LATENT:],
                              lp["face"]["W"][0][3 * LATENT:]], axis=1)
        bu = jnp.concatenate([lp["edge"]["b"][0], lp["node"]["b"][0],
                              lp["face"]["b"][0]])
        uproj = _mm_bias(u, Wu, bu)
        ue_tbl = uproj[:, :LATENT]
        gn_tbl = uproj[:, LATENT:2 * LATENT]
        gf_tbl = uproj[:, 2 * LATENT:]

        if li == 0:
            fef = _dot(f0, Wef)
            ue_tbl = ue_tbl + fef[:, :LATENT] + fef[:, LATENT:]
            tbl = jnp.concatenate([xb, xc], axis=0)
            hpre = _gather_sum(tbl, idxs2)
        else:
            fe, ff = _proj2(face, Wef, 2000)
            tbl = jnp.concatenate([xb, xc, fe, ff], axis=0)
            hpre = _gather_sum(tbl, idxs4)

        e1, ea, eag = _edge_finish(hpre, ea, A, lp["edge"]["W"][1],
                                   ue_tbl, lp["edge"]["b"][1].reshape(1, -1))

        scat = _scatter_sum(e1, idxs4, 2 * (N + F))

        Wn = lp["node"]["W"][0]
        xl, na, xsum = _update(xl, scat, 0, N // NPG,
                               Wn[:LATENT], Wn[LATENT:2 * LATENT],
                               Wn[2 * LATENT:3 * LATENT], gn_tbl,
                               lp["node"]["W"][1],
                               lp["node"]["b"][1].reshape(1, -1), NPG)
        Wf = lp["face"]["W"][0]
        face, fa, fsum = _update(face, scat, 2 * N // FPG, (2 * N + F) // FPG,
                                 Wf[:LATENT], Wf[LATENT:2 * LATENT],
                                 Wf[2 * LATENT:3 * LATENT], gf_tbl,
                                 lp["face"]["W"][1],
                                 lp["face"]["b"][1].reshape(1, -1), FPG)
        Wg = lp["glob"]["W"][0]
        u = _glob_update(u, na, eag, fa, Wg[:LATENT], Wg[LATENT:2 * LATENT],
                         Wg[2 * LATENT:3 * LATENT], Wg[3 * LATENT:],
                         lp["glob"]["b"][0], lp["glob"]["W"][1],
                         lp["glob"]["b"][1])

    r1 = jax.random.uniform(jax.random.key(7), (G, 1), dtype=jnp.float32)
    m1 = ((r1 >= DROPNET) | (num_faces[:, None] == 1)).astype(jnp.float32)
    m2 = ((1.0 - r1) >= DROPNET).astype(jnp.float32)
    pxm = xsum * m1
    pfm = fsum * m2
    Wd = P["decoder"]["W"][0]
    w2p = jnp.pad(P["decoder"]["W"][1], ((0, 0), (0, LATENT - 1)))
    b2p = jnp.pad(P["decoder"]["b"][1], (0, LATENT - 1)).reshape(1, -1)
    outp = _decode(pxm, pfm, Wd[:LATENT], Wd[LATENT:],
                   P["decoder"]["b"][0], w2p, b2p)
    return outp[:, :1]
